# baseline (speedup 1.0000x reference)
import functools

import jax
import jax.numpy as jnp
from jax import lax
from jax.experimental import pallas as pl
from jax.experimental.pallas import tpu as pltpu
from jax.experimental.pallas import tpu_sc as plsc

_NC = 2
_NS = 16
_NW = _NC * _NS
_L = 16

_CHUNK = 40


def _pack_bf16_pairs(a):
    h = a.shape[1] // 2
    lo = lax.bitcast_convert_type(
        a[:, :h].astype(jnp.bfloat16), jnp.uint16).astype(jnp.int32)
    hi = lax.bitcast_convert_type(
        a[:, h:].astype(jnp.bfloat16), jnp.uint16).astype(jnp.int32)
    return (hi << 16) | lo


def _proj_body(x_ref, w1a_ref, w1b_ref, b1_ref, xs_ref, xd_ref):
    xv = x_ref[...]
    xs = jnp.dot(xv, w1a_ref[...], preferred_element_type=jnp.float32)
    xd = (
        jnp.dot(xv, w1b_ref[...], preferred_element_type=jnp.float32)
        + b1_ref[...]
    )
    xs_ref[...] = _pack_bf16_pairs(xs)
    xd_ref[...] = _pack_bf16_pairs(xd)


def _project_nodes(x, w1a, w1b, b1_row):
    n, d = x.shape
    blk = 1000
    grid = n // blk
    return pl.pallas_call(
        _proj_body,
        grid=(grid,),
        in_specs=[
            pl.BlockSpec((blk, d), lambda i: (i, 0)),
            pl.BlockSpec((d, d), lambda i: (0, 0)),
            pl.BlockSpec((d, d), lambda i: (0, 0)),
            pl.BlockSpec((1, d), lambda i: (0, 0)),
        ],
        out_specs=[
            pl.BlockSpec((blk, d // 2), lambda i: (i, 0)),
            pl.BlockSpec((blk, d // 2), lambda i: (i, 0)),
        ],
        out_shape=[
            jax.ShapeDtypeStruct((n, d // 2), jnp.int32),
            jax.ShapeDtypeStruct((n, d // 2), jnp.int32),
        ],
    )(x, w1a, w1b, b1_row)


def _make_sc_edge_kernel(e_total, d):
    epw = e_total // _NW
    nch = epw // _CHUNK
    nwb = d // (2 * _L)
    dw = d // 2
    mesh = plsc.VectorSubcoreMesh(core_axis_name="c", subcore_axis_name="s")

    @functools.partial(
        pl.kernel,
        out_type=jax.ShapeDtypeStruct((e_total,), jnp.float32),
        mesh=mesh,
        scratch_types=[
            pltpu.VMEM((epw,), jnp.int32),
            pltpu.VMEM((epw,), jnp.int32),
            pltpu.VMEM((_CHUNK, dw), jnp.int32),
            pltpu.VMEM((_CHUNK, dw), jnp.int32),
            pltpu.VMEM((_CHUNK, dw), jnp.int32),
            pltpu.VMEM((_CHUNK, dw), jnp.int32),
            pltpu.VMEM((_CHUNK, dw), jnp.int32),
            pltpu.VMEM((_CHUNK, dw), jnp.int32),
            pltpu.VMEM((_CHUNK, dw), jnp.int32),
            pltpu.VMEM((_CHUNK, dw), jnp.int32),
            pltpu.VMEM((epw,), jnp.float32),
            pltpu.VMEM((d + _L,), jnp.float32),
            pltpu.SemaphoreType.DMA,
            pltpu.SemaphoreType.DMA,
            pltpu.SemaphoreType.DMA,
            pltpu.SemaphoreType.DMA,
            pltpu.SemaphoreType.DMA,
            pltpu.SemaphoreType.DMA,
            pltpu.SemaphoreType.DMA,
            pltpu.SemaphoreType.DMA,
        ],
        compiler_params=pltpu.CompilerParams(
            needs_layout_passes=False, use_tc_tiling_on_sc=False),
    )
    def sc_edge_kernel(xs_hbm, xd_hbm, src_hbm, dst_hbm, p_hbm, out_hbm,
                       idx_s, idx_d, rs0, rs1, rs2, rs3, rd0, rd1, rd2, rd3,
                       out_v, p_v, ss0, ss1, ss2, ss3, sd0, sd1, sd2, sd3):
        wid = lax.axis_index("s") * _NC + lax.axis_index("c")
        base = wid * epw

        pltpu.sync_copy(src_hbm.at[pl.ds(base, epw)], idx_s)
        pltpu.sync_copy(dst_hbm.at[pl.ds(base, epw)], idx_d)
        pltpu.sync_copy(p_hbm, p_v)

        w2v = [p_v[pl.ds(_L * j, _L)] for j in range(2 * nwb)]
        bias = p_v[pl.ds(d, _L)]
        lane = lax.iota(jnp.int32, _L)
        xperm = {sh: jnp.bitwise_xor(lane, sh) for sh in (8, 4, 2, 1)}
        xmask = {sh: (lane & sh) == 0 for sh in (8, 4, 2, 1)}
        perm_e = (
            ((lane >> 3) & 1)
            | (((lane >> 2) & 1) << 1)
            | (((lane >> 1) & 1) << 2)
        )
        zero_v = jnp.zeros((_L,), jnp.float32)
        zero_b = jnp.zeros((2 * _L,), jnp.bfloat16)
        eight_v = jnp.full((_L,), 8, dtype=jnp.int32)
        himask = jnp.full((_L,), -65536, dtype=jnp.int32)

        def xg(v, sh):
            return v.at[xperm[sh]].get(mode="promise_in_bounds")

        rs = [rs0, rs1, rs2, rs3]
        rd = [rd0, rd1, rd2, rd3]
        sems_s = [ss0, ss1, ss2, ss3]
        sems_d = [sd0, sd1, sd2, sd3]
        nbuf = 4

        def gather(chunk, b):
            return (
                pltpu.make_async_copy(
                    xs_hbm.at[idx_s.at[pl.ds(chunk * _CHUNK, _CHUNK)]],
                    rs[b], sems_s[b]),
                pltpu.make_async_copy(
                    xd_hbm.at[idx_d.at[pl.ds(chunk * _CHUNK, _CHUNK)]],
                    rd[b], sems_d[b]),
            )

        def issue(chunk, b):
            cs, cd = gather(chunk, b)
            cs.start()
            cd.start()

        for b in range(nbuf):
            issue(b, b)

        def consume(chunk, b, do_issue):
            cs, cd = gather(chunk, b)
            cs.wait()
            cd.wait()

            def block_body(k, pos, b=b):
                vs = []
                for t in range(8):
                    e = k * 8 + t
                    accs = [bias, zero_v]
                    for q in range(nwb):
                        ws = rs[b][e, pl.ds(_L * q, _L)]
                        wd = rd[b][e, pl.ds(_L * q, _L)]
                        bsum = (plsc.bitcast(ws, jnp.bfloat16)
                                + plsc.bitcast(wd, jnp.bfloat16))
                        brel = jnp.maximum(bsum, zero_b)
                        wi = plsc.bitcast(brel, jnp.int32)
                        r_lo = plsc.bitcast(wi << 16, jnp.float32)
                        r_hi = plsc.bitcast(wi & himask, jnp.float32)
                        accs[0] = accs[0] + r_lo * w2v[q]
                        accs[1] = accs[1] + r_hi * w2v[q + nwb]
                    vs.append(accs[0] + accs[1])
                for sh in (8, 4, 2):
                    m = xmask[sh]
                    vs = [
                        jnp.where(m, u + xg(u, sh), xg(v + xg(v, sh), sh))
                        for u, v in zip(vs[0::2], vs[1::2])
                    ]
                f = vs[0]
                f = f + xg(f, 1)
                plsc.store_scatter(out_v, [pos], f, mask=xmask[1])
                return pos + eight_v

            pos0 = perm_e + (chunk * _CHUNK)
            lax.fori_loop(0, _CHUNK // 8, block_body, pos0, unroll=1)

            if do_issue:
                @pl.when(chunk + nbuf < nch)
                def _(b=b, chunk=chunk):
                    issue(chunk + nbuf, b)

        def outer(i, carry):
            c4 = i * nbuf
            for b in range(nbuf):
                consume(c4 + b, b, True)
            return carry

        lax.fori_loop(0, nch // nbuf, outer, 0)
        for r in range(nch % nbuf):
            consume((nch // nbuf) * nbuf + r, r, False)

        pltpu.sync_copy(out_v, out_hbm.at[pl.ds(base, epw)])

    return sc_edge_kernel


@jax.jit
def kernel(x, edge_index, W1, b1, W2, b2):
    n, d = x.shape
    e_total = edge_index.shape[1]

    w1a = W1[:d]
    w1b = W1[d:]
    xs, xd = _project_nodes(x, w1a, w1b, b1.reshape(1, d))

    p = jnp.concatenate(
        [W2[:, 0], jnp.full((_L,), b2[0] / _L, dtype=jnp.float32)])

    src = edge_index[0]
    dst = edge_index[1]

    sc_kernel = _make_sc_edge_kernel(e_total, d)
    return sc_kernel(xs, xd, src, dst, p)


# speedup vs baseline: 1.0615x; 1.0615x over previous
import functools

import jax
import jax.numpy as jnp
from jax import lax
from jax.experimental import pallas as pl
from jax.experimental.pallas import tpu as pltpu
from jax.experimental.pallas import tpu_sc as plsc

_NC = 2
_NS = 16
_NW = _NC * _NS
_L = 16

_CHUNK = 40


def _pack_bf16_pairs(a):
    h = a.shape[1] // 2
    lo = lax.bitcast_convert_type(
        a[:, :h].astype(jnp.bfloat16), jnp.uint16).astype(jnp.int32)
    hi = lax.bitcast_convert_type(
        a[:, h:].astype(jnp.bfloat16), jnp.uint16).astype(jnp.int32)
    return (hi << 16) | lo


def _proj_body(x_ref, w1a_ref, w1b_ref, b1_ref, xs_ref, xd_ref):
    xv = x_ref[...]
    xs = jnp.dot(xv, w1a_ref[...], preferred_element_type=jnp.float32)
    xd = (
        jnp.dot(xv, w1b_ref[...], preferred_element_type=jnp.float32)
        + b1_ref[...]
    )
    xs_ref[...] = _pack_bf16_pairs(xs)
    xd_ref[...] = _pack_bf16_pairs(xd)


def _project_nodes(x, w1a, w1b, b1_row):
    n, d = x.shape
    blk = 1000
    grid = n // blk
    return pl.pallas_call(
        _proj_body,
        grid=(grid,),
        in_specs=[
            pl.BlockSpec((blk, d), lambda i: (i, 0)),
            pl.BlockSpec((d, d), lambda i: (0, 0)),
            pl.BlockSpec((d, d), lambda i: (0, 0)),
            pl.BlockSpec((1, d), lambda i: (0, 0)),
        ],
        out_specs=[
            pl.BlockSpec((blk, d // 2), lambda i: (i, 0)),
            pl.BlockSpec((blk, d // 2), lambda i: (i, 0)),
        ],
        out_shape=[
            jax.ShapeDtypeStruct((n, d // 2), jnp.int32),
            jax.ShapeDtypeStruct((n, d // 2), jnp.int32),
        ],
    )(x, w1a, w1b, b1_row)


def _make_sc_edge_kernel(e_total, d):
    epw = e_total // _NW
    nch = epw // _CHUNK
    nwb = d // (2 * _L)
    dw = d // 2
    mesh = plsc.VectorSubcoreMesh(core_axis_name="c", subcore_axis_name="s")

    @functools.partial(
        pl.kernel,
        out_type=jax.ShapeDtypeStruct((e_total,), jnp.float32),
        mesh=mesh,
        scratch_types=[
            pltpu.VMEM((epw,), jnp.int32),
            pltpu.VMEM((epw,), jnp.int32),
            pltpu.VMEM((_CHUNK, dw), jnp.int32),
            pltpu.VMEM((_CHUNK, dw), jnp.int32),
            pltpu.VMEM((_CHUNK, dw), jnp.int32),
            pltpu.VMEM((_CHUNK, dw), jnp.int32),
            pltpu.VMEM((epw,), jnp.float32),
            pltpu.VMEM((d + _L,), jnp.float32),
            pltpu.SemaphoreType.DMA,
            pltpu.SemaphoreType.DMA,
            pltpu.SemaphoreType.DMA,
            pltpu.SemaphoreType.DMA,
        ],
        compiler_params=pltpu.CompilerParams(
            needs_layout_passes=False, use_tc_tiling_on_sc=False),
    )
    def sc_edge_kernel(xs_hbm, xd_hbm, src_hbm, dst_hbm, p_hbm, out_hbm,
                       idx_s, idx_d, rs0, rs1, rd0, rd1,
                       out_v, p_v, ss0, ss1, sd0, sd1):
        wid = lax.axis_index("s") * _NC + lax.axis_index("c")
        base = wid * epw

        pltpu.sync_copy(src_hbm.at[pl.ds(base, epw)], idx_s)
        pltpu.sync_copy(dst_hbm.at[pl.ds(base, epw)], idx_d)
        pltpu.sync_copy(p_hbm, p_v)

        w2v = [p_v[pl.ds(_L * j, _L)] for j in range(2 * nwb)]
        bias = p_v[pl.ds(d, _L)]
        lane = lax.iota(jnp.int32, _L)
        xperm = {sh: jnp.bitwise_xor(lane, sh) for sh in (8, 4, 2, 1)}
        xmask = {sh: (lane & sh) == 0 for sh in (8, 4, 2, 1)}
        perm_e = (
            ((lane >> 3) & 1)
            | (((lane >> 2) & 1) << 1)
            | (((lane >> 1) & 1) << 2)
        )
        zero_v = jnp.zeros((_L,), jnp.float32)
        zero_b = jnp.zeros((2 * _L,), jnp.bfloat16)
        eight_v = jnp.full((_L,), 8, dtype=jnp.int32)
        himask = jnp.full((_L,), -65536, dtype=jnp.int32)

        def xg(v, sh):
            return v.at[xperm[sh]].get(mode="promise_in_bounds")

        rs = [rs0, rs1]
        rd = [rd0, rd1]
        sems_s = [ss0, ss1]
        sems_d = [sd0, sd1]
        nbuf = 2

        def gather(chunk, b):
            return (
                pltpu.make_async_copy(
                    xs_hbm.at[idx_s.at[pl.ds(chunk * _CHUNK, _CHUNK)]],
                    rs[b], sems_s[b]),
                pltpu.make_async_copy(
                    xd_hbm.at[idx_d.at[pl.ds(chunk * _CHUNK, _CHUNK)]],
                    rd[b], sems_d[b]),
            )

        def issue(chunk, b):
            cs, cd = gather(chunk, b)
            cs.start()
            cd.start()

        for b in range(nbuf):
            issue(b, b)

        def consume(chunk, b, do_issue):
            cs, cd = gather(chunk, b)
            cs.wait()
            cd.wait()

            def block_body(k, pos, b=b):
                vs = []
                for t in range(8):
                    e = k * 8 + t
                    accs = [bias, zero_v]
                    for q in range(nwb):
                        ws = rs[b][e, pl.ds(_L * q, _L)]
                        wd = rd[b][e, pl.ds(_L * q, _L)]
                        bsum = (plsc.bitcast(ws, jnp.bfloat16)
                                + plsc.bitcast(wd, jnp.bfloat16))
                        brel = jnp.maximum(bsum, zero_b)
                        wi = plsc.bitcast(brel, jnp.int32)
                        r_lo = plsc.bitcast(wi << 16, jnp.float32)
                        r_hi = plsc.bitcast(wi & himask, jnp.float32)
                        accs[0] = accs[0] + r_lo * w2v[q]
                        accs[1] = accs[1] + r_hi * w2v[q + nwb]
                    vs.append(accs[0] + accs[1])
                for sh in (8, 4, 2):
                    m = xmask[sh]
                    vs = [
                        jnp.where(m, u + xg(u, sh), xg(v + xg(v, sh), sh))
                        for u, v in zip(vs[0::2], vs[1::2])
                    ]
                f = vs[0]
                f = f + xg(f, 1)
                plsc.store_scatter(out_v, [pos], f, mask=xmask[1])
                return pos + eight_v

            pos0 = perm_e + (chunk * _CHUNK)
            lax.fori_loop(0, _CHUNK // 8, block_body, pos0, unroll=1)

            if do_issue:
                @pl.when(chunk + nbuf < nch)
                def _(b=b, chunk=chunk):
                    issue(chunk + nbuf, b)

        def outer(i, carry):
            c4 = i * nbuf
            for b in range(nbuf):
                consume(c4 + b, b, True)
            return carry

        lax.fori_loop(0, nch // nbuf, outer, 0)
        for r in range(nch % nbuf):
            consume((nch // nbuf) * nbuf + r, r, False)

        pltpu.sync_copy(out_v, out_hbm.at[pl.ds(base, epw)])

    return sc_edge_kernel


@jax.jit
def kernel(x, edge_index, W1, b1, W2, b2):
    n, d = x.shape
    e_total = edge_index.shape[1]

    w1a = W1[:d]
    w1b = W1[d:]
    xs, xd = _project_nodes(x, w1a, w1b, b1.reshape(1, d))

    p = jnp.concatenate(
        [W2[:, 0], jnp.full((_L,), b2[0] / _L, dtype=jnp.float32)])

    src = edge_index[0]
    dst = edge_index[1]

    sc_kernel = _make_sc_edge_kernel(e_total, d)
    return sc_kernel(xs, xd, src, dst, p)


# speedup vs baseline: 1.2819x; 1.2077x over previous
import functools

import jax
import jax.numpy as jnp
from jax import lax
from jax.experimental import pallas as pl
from jax.experimental.pallas import tpu as pltpu
from jax.experimental.pallas import tpu_sc as plsc

_NC = 2
_NS = 16
_NW = _NC * _NS
_L = 16

_CHUNK = 80


def _pack_bf16_pairs(a):
    h = a.shape[1] // 2
    lo = lax.bitcast_convert_type(
        a[:, :h].astype(jnp.bfloat16), jnp.uint16).astype(jnp.int32)
    hi = lax.bitcast_convert_type(
        a[:, h:].astype(jnp.bfloat16), jnp.uint16).astype(jnp.int32)
    return (hi << 16) | lo


def _proj_body(x_ref, w1a_ref, w1b_ref, b1_ref, xs_ref, xd_ref):
    xv = x_ref[...]
    xs = jnp.dot(xv, w1a_ref[...], preferred_element_type=jnp.float32)
    xd = (
        jnp.dot(xv, w1b_ref[...], preferred_element_type=jnp.float32)
        + b1_ref[...]
    )
    xs_ref[...] = _pack_bf16_pairs(xs)
    xd_ref[...] = _pack_bf16_pairs(xd)


def _project_nodes(x, w1a, w1b, b1_row):
    n, d = x.shape
    blk = 1000
    grid = n // blk
    return pl.pallas_call(
        _proj_body,
        grid=(grid,),
        in_specs=[
            pl.BlockSpec((blk, d), lambda i: (i, 0)),
            pl.BlockSpec((d, d), lambda i: (0, 0)),
            pl.BlockSpec((d, d), lambda i: (0, 0)),
            pl.BlockSpec((1, d), lambda i: (0, 0)),
        ],
        out_specs=[
            pl.BlockSpec((blk, d // 2), lambda i: (i, 0)),
            pl.BlockSpec((blk, d // 2), lambda i: (i, 0)),
        ],
        out_shape=[
            jax.ShapeDtypeStruct((n, d // 2), jnp.int32),
            jax.ShapeDtypeStruct((n, d // 2), jnp.int32),
        ],
    )(x, w1a, w1b, b1_row)


def _make_sc_edge_kernel(e_total, d):
    epw = e_total // _NW
    nch = epw // _CHUNK
    nwb = d // (2 * _L)
    dw = d // 2
    mesh = plsc.VectorSubcoreMesh(core_axis_name="c", subcore_axis_name="s")

    @functools.partial(
        pl.kernel,
        out_type=jax.ShapeDtypeStruct((e_total,), jnp.float32),
        mesh=mesh,
        scratch_types=[
            pltpu.VMEM((epw,), jnp.int32),
            pltpu.VMEM((epw,), jnp.int32),
            pltpu.VMEM((_CHUNK, dw), jnp.int32),
            pltpu.VMEM((_CHUNK, dw), jnp.int32),
            pltpu.VMEM((_CHUNK, dw), jnp.int32),
            pltpu.VMEM((_CHUNK, dw), jnp.int32),
            pltpu.VMEM((epw,), jnp.float32),
            pltpu.VMEM((d + _L,), jnp.float32),
            pltpu.SemaphoreType.DMA,
            pltpu.SemaphoreType.DMA,
            pltpu.SemaphoreType.DMA,
            pltpu.SemaphoreType.DMA,
        ],
        compiler_params=pltpu.CompilerParams(
            needs_layout_passes=False, use_tc_tiling_on_sc=False),
    )
    def sc_edge_kernel(xs_hbm, xd_hbm, src_hbm, dst_hbm, p_hbm, out_hbm,
                       idx_s, idx_d, rs0, rs1, rd0, rd1,
                       out_v, p_v, ss0, ss1, sd0, sd1):
        wid = lax.axis_index("s") * _NC + lax.axis_index("c")
        base = wid * epw

        pltpu.sync_copy(src_hbm.at[pl.ds(base, epw)], idx_s)
        pltpu.sync_copy(dst_hbm.at[pl.ds(base, epw)], idx_d)
        pltpu.sync_copy(p_hbm, p_v)

        w2v = [p_v[pl.ds(_L * j, _L)] for j in range(2 * nwb)]
        bias = p_v[pl.ds(d, _L)]
        lane = lax.iota(jnp.int32, _L)
        xperm = {sh: jnp.bitwise_xor(lane, sh) for sh in (8, 4, 2, 1)}
        xmask = {sh: (lane & sh) == 0 for sh in (8, 4, 2, 1)}
        perm_e = (
            ((lane >> 3) & 1)
            | (((lane >> 2) & 1) << 1)
            | (((lane >> 1) & 1) << 2)
        )
        zero_v = jnp.zeros((_L,), jnp.float32)
        zero_b = jnp.zeros((2 * _L,), jnp.bfloat16)
        eight_v = jnp.full((_L,), 8, dtype=jnp.int32)
        himask = jnp.full((_L,), -65536, dtype=jnp.int32)

        def xg(v, sh):
            return v.at[xperm[sh]].get(mode="promise_in_bounds")

        rs = [rs0, rs1]
        rd = [rd0, rd1]
        sems_s = [ss0, ss1]
        sems_d = [sd0, sd1]
        nbuf = 2

        def gather(chunk, b):
            return (
                pltpu.make_async_copy(
                    xs_hbm.at[idx_s.at[pl.ds(chunk * _CHUNK, _CHUNK)]],
                    rs[b], sems_s[b]),
                pltpu.make_async_copy(
                    xd_hbm.at[idx_d.at[pl.ds(chunk * _CHUNK, _CHUNK)]],
                    rd[b], sems_d[b]),
            )

        def issue(chunk, b):
            cs, cd = gather(chunk, b)
            cs.start()
            cd.start()

        for b in range(nbuf):
            issue(b, b)

        def consume(chunk, b, do_issue):
            cs, cd = gather(chunk, b)
            cs.wait()
            cd.wait()

            def block_body(k, pos, b=b):
                vs = []
                for t in range(8):
                    e = k * 8 + t
                    accs = [bias, zero_v]
                    for q in range(nwb):
                        ws = rs[b][e, pl.ds(_L * q, _L)]
                        wd = rd[b][e, pl.ds(_L * q, _L)]
                        bsum = (plsc.bitcast(ws, jnp.bfloat16)
                                + plsc.bitcast(wd, jnp.bfloat16))
                        brel = jnp.maximum(bsum, zero_b)
                        wi = plsc.bitcast(brel, jnp.int32)
                        r_lo = plsc.bitcast(wi << 16, jnp.float32)
                        r_hi = plsc.bitcast(wi & himask, jnp.float32)
                        accs[0] = accs[0] + r_lo * w2v[q]
                        accs[1] = accs[1] + r_hi * w2v[q + nwb]
                    vs.append(accs[0] + accs[1])
                for sh in (8, 4, 2):
                    m = xmask[sh]
                    vs = [
                        jnp.where(m, u + xg(u, sh), xg(v + xg(v, sh), sh))
                        for u, v in zip(vs[0::2], vs[1::2])
                    ]
                f = vs[0]
                f = f + xg(f, 1)
                plsc.store_scatter(out_v, [pos], f, mask=xmask[1])
                return pos + eight_v

            pos0 = perm_e + (chunk * _CHUNK)
            lax.fori_loop(0, _CHUNK // 8, block_body, pos0, unroll=1)

            if do_issue:
                @pl.when(chunk + nbuf < nch)
                def _(b=b, chunk=chunk):
                    issue(chunk + nbuf, b)

        def outer(i, carry):
            c4 = i * nbuf
            for b in range(nbuf):
                consume(c4 + b, b, True)
            return carry

        lax.fori_loop(0, nch // nbuf, outer, 0)
        for r in range(nch % nbuf):
            consume((nch // nbuf) * nbuf + r, r, False)

        pltpu.sync_copy(out_v, out_hbm.at[pl.ds(base, epw)])

    return sc_edge_kernel


@jax.jit
def kernel(x, edge_index, W1, b1, W2, b2):
    n, d = x.shape
    e_total = edge_index.shape[1]

    w1a = W1[:d]
    w1b = W1[d:]
    xs, xd = _project_nodes(x, w1a, w1b, b1.reshape(1, d))

    p = jnp.concatenate(
        [W2[:, 0], jnp.full((_L,), b2[0] / _L, dtype=jnp.float32)])

    src = edge_index[0]
    dst = edge_index[1]

    sc_kernel = _make_sc_edge_kernel(e_total, d)
    return sc_kernel(xs, xd, src, dst, p)


# speedup vs baseline: 1.3608x; 1.0616x over previous
import functools

import jax
import jax.numpy as jnp
from jax import lax
from jax.experimental import pallas as pl
from jax.experimental.pallas import tpu as pltpu
from jax.experimental.pallas import tpu_sc as plsc

_NC = 2
_NS = 16
_NW = _NC * _NS
_L = 16

_CHUNK = 80


def _pack_bf16_pairs(a):
    h = a.shape[1] // 2
    lo = lax.bitcast_convert_type(
        a[:, :h].astype(jnp.bfloat16), jnp.uint16).astype(jnp.int32)
    hi = lax.bitcast_convert_type(
        a[:, h:].astype(jnp.bfloat16), jnp.uint16).astype(jnp.int32)
    return (hi << 16) | lo


def _proj_body(x_ref, w1a_ref, w1b_ref, b1_ref, xs_ref, xd_ref):
    xv = x_ref[...]
    xs = jnp.dot(xv, w1a_ref[...], preferred_element_type=jnp.float32)
    xd = (
        jnp.dot(xv, w1b_ref[...], preferred_element_type=jnp.float32)
        + b1_ref[...]
    )
    xs_ref[...] = _pack_bf16_pairs(xs)
    xd_ref[...] = _pack_bf16_pairs(xd)


def _project_nodes(x, w1a, w1b, b1_row):
    n, d = x.shape
    blk = 1000
    grid = n // blk
    return pl.pallas_call(
        _proj_body,
        grid=(grid,),
        in_specs=[
            pl.BlockSpec((blk, d), lambda i: (i, 0)),
            pl.BlockSpec((d, d), lambda i: (0, 0)),
            pl.BlockSpec((d, d), lambda i: (0, 0)),
            pl.BlockSpec((1, d), lambda i: (0, 0)),
        ],
        out_specs=[
            pl.BlockSpec((blk, d // 2), lambda i: (i, 0)),
            pl.BlockSpec((blk, d // 2), lambda i: (i, 0)),
        ],
        out_shape=[
            jax.ShapeDtypeStruct((n, d // 2), jnp.int32),
            jax.ShapeDtypeStruct((n, d // 2), jnp.int32),
        ],
    )(x, w1a, w1b, b1_row)


def _make_sc_edge_kernel(e_total, d):
    epw = e_total // _NW
    nch = epw // _CHUNK
    nwb = d // (2 * _L)
    dw = d // 2
    mesh = plsc.VectorSubcoreMesh(core_axis_name="c", subcore_axis_name="s")

    @functools.partial(
        pl.kernel,
        out_type=jax.ShapeDtypeStruct((e_total,), jnp.float32),
        mesh=mesh,
        scratch_types=[
            pltpu.VMEM((epw,), jnp.int32),
            pltpu.VMEM((epw,), jnp.int32),
            pltpu.VMEM((_CHUNK, dw), jnp.int32),
            pltpu.VMEM((_CHUNK, dw), jnp.int32),
            pltpu.VMEM((_CHUNK, dw), jnp.int32),
            pltpu.VMEM((_CHUNK, dw), jnp.int32),
            pltpu.VMEM((epw,), jnp.float32),
            pltpu.VMEM((d + _L,), jnp.float32),
            pltpu.SemaphoreType.DMA,
            pltpu.SemaphoreType.DMA,
            pltpu.SemaphoreType.DMA,
            pltpu.SemaphoreType.DMA,
        ],
        compiler_params=pltpu.CompilerParams(
            needs_layout_passes=False, use_tc_tiling_on_sc=False),
    )
    def sc_edge_kernel(xs_hbm, xd_hbm, ei_hbm, p_hbm, out_hbm,
                       idx_s, idx_d, rs0, rs1, rd0, rd1,
                       out_v, p_v, ss0, ss1, sd0, sd1):
        wid = lax.axis_index("s") * _NC + lax.axis_index("c")
        base = wid * epw

        pltpu.sync_copy(ei_hbm.at[0, pl.ds(base, epw)], idx_s)
        pltpu.sync_copy(ei_hbm.at[1, pl.ds(base, epw)], idx_d)
        pltpu.sync_copy(p_hbm, p_v)

        w2v = [p_v[pl.ds(_L * j, _L)] for j in range(2 * nwb)]
        bias = p_v[pl.ds(d, _L)]
        lane = lax.iota(jnp.int32, _L)
        xperm = {sh: jnp.bitwise_xor(lane, sh) for sh in (8, 4, 2, 1)}
        xmask = {sh: (lane & sh) == 0 for sh in (8, 4, 2, 1)}
        perm_e = (
            ((lane >> 3) & 1)
            | (((lane >> 2) & 1) << 1)
            | (((lane >> 1) & 1) << 2)
        )
        zero_v = jnp.zeros((_L,), jnp.float32)
        zero_b = jnp.zeros((2 * _L,), jnp.bfloat16)
        eight_v = jnp.full((_L,), 8, dtype=jnp.int32)
        himask = jnp.full((_L,), -65536, dtype=jnp.int32)

        def xg(v, sh):
            return v.at[xperm[sh]].get(mode="promise_in_bounds")

        rs = [rs0, rs1]
        rd = [rd0, rd1]
        sems_s = [ss0, ss1]
        sems_d = [sd0, sd1]
        nbuf = 2

        def gather(chunk, b):
            return (
                pltpu.make_async_copy(
                    xs_hbm.at[idx_s.at[pl.ds(chunk * _CHUNK, _CHUNK)]],
                    rs[b], sems_s[b]),
                pltpu.make_async_copy(
                    xd_hbm.at[idx_d.at[pl.ds(chunk * _CHUNK, _CHUNK)]],
                    rd[b], sems_d[b]),
            )

        def issue(chunk, b):
            cs, cd = gather(chunk, b)
            cs.start()
            cd.start()

        for b in range(nbuf):
            issue(b, b)

        def consume(chunk, b, do_issue):
            cs, cd = gather(chunk, b)
            cs.wait()
            cd.wait()

            def block_body(k, pos, b=b):
                vs = []
                for t in range(8):
                    e = k * 8 + t
                    accs = [bias, zero_v]
                    for q in range(nwb):
                        ws = rs[b][e, pl.ds(_L * q, _L)]
                        wd = rd[b][e, pl.ds(_L * q, _L)]
                        bsum = (plsc.bitcast(ws, jnp.bfloat16)
                                + plsc.bitcast(wd, jnp.bfloat16))
                        brel = jnp.maximum(bsum, zero_b)
                        wi = plsc.bitcast(brel, jnp.int32)
                        r_lo = plsc.bitcast(wi << 16, jnp.float32)
                        r_hi = plsc.bitcast(wi & himask, jnp.float32)
                        accs[0] = accs[0] + r_lo * w2v[q]
                        accs[1] = accs[1] + r_hi * w2v[q + nwb]
                    vs.append(accs[0] + accs[1])
                for sh in (8, 4, 2):
                    m = xmask[sh]
                    vs = [
                        jnp.where(m, u + xg(u, sh), xg(v + xg(v, sh), sh))
                        for u, v in zip(vs[0::2], vs[1::2])
                    ]
                f = vs[0]
                f = f + xg(f, 1)
                plsc.store_scatter(out_v, [pos], f, mask=xmask[1])
                return pos + eight_v

            pos0 = perm_e + (chunk * _CHUNK)
            lax.fori_loop(0, _CHUNK // 8, block_body, pos0, unroll=1)

            if do_issue:
                @pl.when(chunk + nbuf < nch)
                def _(b=b, chunk=chunk):
                    issue(chunk + nbuf, b)

        def outer(i, carry):
            c4 = i * nbuf
            for b in range(nbuf):
                consume(c4 + b, b, True)
            return carry

        lax.fori_loop(0, nch // nbuf, outer, 0)
        for r in range(nch % nbuf):
            consume((nch // nbuf) * nbuf + r, r, False)

        pltpu.sync_copy(out_v, out_hbm.at[pl.ds(base, epw)])

    return sc_edge_kernel


@jax.jit
def kernel(x, edge_index, W1, b1, W2, b2):
    n, d = x.shape
    e_total = edge_index.shape[1]

    w1a = W1[:d]
    w1b = W1[d:]
    xs, xd = _project_nodes(x, w1a, w1b, b1.reshape(1, d))

    p = jnp.concatenate(
        [W2[:, 0], jnp.full((_L,), b2[0] / _L, dtype=jnp.float32)])

    sc_kernel = _make_sc_edge_kernel(e_total, d)
    return sc_kernel(xs, xd, edge_index, p)


# speedup vs baseline: 1.4679x; 1.0787x over previous
import functools

import jax
import jax.numpy as jnp
from jax import lax
from jax.experimental import pallas as pl
from jax.experimental.pallas import tpu as pltpu
from jax.experimental.pallas import tpu_sc as plsc

_NC = 2
_NS = 16
_NW = _NC * _NS
_L = 16

_CHUNK = 128


def _pack_bf16_pairs(a):
    h = a.shape[1] // 2
    lo = lax.bitcast_convert_type(
        a[:, :h].astype(jnp.bfloat16), jnp.uint16).astype(jnp.int32)
    hi = lax.bitcast_convert_type(
        a[:, h:].astype(jnp.bfloat16), jnp.uint16).astype(jnp.int32)
    return (hi << 16) | lo


def _proj_body(x_ref, w1a_ref, w1b_ref, b1_ref, xs_ref, xd_ref):
    xv = x_ref[...]
    xs = jnp.dot(xv, w1a_ref[...], preferred_element_type=jnp.float32)
    xd = (
        jnp.dot(xv, w1b_ref[...], preferred_element_type=jnp.float32)
        + b1_ref[...]
    )
    xs_ref[...] = _pack_bf16_pairs(xs)
    xd_ref[...] = _pack_bf16_pairs(xd)


def _project_nodes(x, w1a, w1b, b1_row):
    n, d = x.shape
    blk = 1000
    grid = n // blk
    return pl.pallas_call(
        _proj_body,
        grid=(grid,),
        in_specs=[
            pl.BlockSpec((blk, d), lambda i: (i, 0)),
            pl.BlockSpec((d, d), lambda i: (0, 0)),
            pl.BlockSpec((d, d), lambda i: (0, 0)),
            pl.BlockSpec((1, d), lambda i: (0, 0)),
        ],
        out_specs=[
            pl.BlockSpec((blk, d // 2), lambda i: (i, 0)),
            pl.BlockSpec((blk, d // 2), lambda i: (i, 0)),
        ],
        out_shape=[
            jax.ShapeDtypeStruct((n, d // 2), jnp.int32),
            jax.ShapeDtypeStruct((n, d // 2), jnp.int32),
        ],
    )(x, w1a, w1b, b1_row)


def _make_sc_edge_kernel(e_total, d):
    epw = e_total // _NW
    nch = epw // _CHUNK
    tail = epw - nch * _CHUNK
    nwb = d // (2 * _L)
    dw = d // 2
    mesh = plsc.VectorSubcoreMesh(core_axis_name="c", subcore_axis_name="s")

    @functools.partial(
        pl.kernel,
        out_type=jax.ShapeDtypeStruct((e_total,), jnp.float32),
        mesh=mesh,
        scratch_types=[
            pltpu.VMEM((epw,), jnp.int32),
            pltpu.VMEM((epw,), jnp.int32),
            pltpu.VMEM((_CHUNK, dw), jnp.int32),
            pltpu.VMEM((_CHUNK, dw), jnp.int32),
            pltpu.VMEM((_CHUNK, dw), jnp.int32),
            pltpu.VMEM((_CHUNK, dw), jnp.int32),
            pltpu.VMEM((epw,), jnp.float32),
            pltpu.VMEM((d + _L,), jnp.float32),
            pltpu.SemaphoreType.DMA,
            pltpu.SemaphoreType.DMA,
            pltpu.SemaphoreType.DMA,
            pltpu.SemaphoreType.DMA,
        ],
        compiler_params=pltpu.CompilerParams(
            needs_layout_passes=False, use_tc_tiling_on_sc=False),
    )
    def sc_edge_kernel(xs_hbm, xd_hbm, ei_hbm, p_hbm, out_hbm,
                       idx_s, idx_d, rs0, rs1, rd0, rd1,
                       out_v, p_v, ss0, ss1, sd0, sd1):
        wid = lax.axis_index("s") * _NC + lax.axis_index("c")
        base = wid * epw

        pltpu.sync_copy(ei_hbm.at[0, pl.ds(base, epw)], idx_s)
        pltpu.sync_copy(ei_hbm.at[1, pl.ds(base, epw)], idx_d)
        pltpu.sync_copy(p_hbm, p_v)

        w2v = [p_v[pl.ds(_L * j, _L)] for j in range(2 * nwb)]
        bias = p_v[pl.ds(d, _L)]
        lane = lax.iota(jnp.int32, _L)
        xperm = {sh: jnp.bitwise_xor(lane, sh) for sh in (8, 4, 2, 1)}
        xmask = {sh: (lane & sh) == 0 for sh in (8, 4, 2, 1)}
        perm_e = (
            ((lane >> 3) & 1)
            | (((lane >> 2) & 1) << 1)
            | (((lane >> 1) & 1) << 2)
        )
        zero_v = jnp.zeros((_L,), jnp.float32)
        zero_b = jnp.zeros((2 * _L,), jnp.bfloat16)
        eight_v = jnp.full((_L,), 8, dtype=jnp.int32)
        himask = jnp.full((_L,), -65536, dtype=jnp.int32)

        def xg(v, sh):
            return v.at[xperm[sh]].get(mode="promise_in_bounds")

        rs = [rs0, rs1]
        rd = [rd0, rd1]
        sems_s = [ss0, ss1]
        sems_d = [sd0, sd1]
        nbuf = 2

        def gather(chunk, b):
            return (
                pltpu.make_async_copy(
                    xs_hbm.at[idx_s.at[pl.ds(chunk * _CHUNK, _CHUNK)]],
                    rs[b], sems_s[b]),
                pltpu.make_async_copy(
                    xd_hbm.at[idx_d.at[pl.ds(chunk * _CHUNK, _CHUNK)]],
                    rd[b], sems_d[b]),
            )

        def issue(chunk, b):
            cs, cd = gather(chunk, b)
            cs.start()
            cd.start()

        for b in range(nbuf):
            issue(b, b)

        def block_body_for(b):
            def block_body(k, pos):
                vs = []
                for t in range(8):
                    e = k * 8 + t
                    accs = [bias, zero_v]
                    for q in range(nwb):
                        ws = rs[b][e, pl.ds(_L * q, _L)]
                        wd = rd[b][e, pl.ds(_L * q, _L)]
                        bsum = (plsc.bitcast(ws, jnp.bfloat16)
                                + plsc.bitcast(wd, jnp.bfloat16))
                        brel = jnp.maximum(bsum, zero_b)
                        wi = plsc.bitcast(brel, jnp.int32)
                        r_lo = plsc.bitcast(wi << 16, jnp.float32)
                        r_hi = plsc.bitcast(wi & himask, jnp.float32)
                        accs[0] = accs[0] + r_lo * w2v[q]
                        accs[1] = accs[1] + r_hi * w2v[q + nwb]
                    vs.append(accs[0] + accs[1])
                for sh in (8, 4, 2):
                    m = xmask[sh]
                    vs = [
                        jnp.where(m, u + xg(u, sh), xg(v + xg(v, sh), sh))
                        for u, v in zip(vs[0::2], vs[1::2])
                    ]
                f = vs[0]
                f = f + xg(f, 1)
                plsc.store_scatter(out_v, [pos], f, mask=xmask[1])
                return pos + eight_v

            return block_body

        def consume(chunk, b, do_issue):
            cs, cd = gather(chunk, b)
            cs.wait()
            cd.wait()

            pos0 = perm_e + (chunk * _CHUNK)
            lax.fori_loop(0, _CHUNK // 8, block_body_for(b), pos0, unroll=1)

            if do_issue:
                @pl.when(chunk + nbuf < nch)
                def _(b=b, chunk=chunk):
                    issue(chunk + nbuf, b)

        def outer(i, carry):
            c4 = i * nbuf
            for b in range(nbuf):
                consume(c4 + b, b, True)
            return carry

        lax.fori_loop(0, nch // nbuf, outer, 0)
        for r in range(nch % nbuf):
            consume((nch // nbuf) * nbuf + r, r, False)

        if tail:
            off = nch * _CHUNK
            tcs = pltpu.make_async_copy(
                xs_hbm.at[idx_s.at[pl.ds(off, tail)]],
                rs[0].at[pl.ds(0, tail)], sems_s[0])
            tcd = pltpu.make_async_copy(
                xd_hbm.at[idx_d.at[pl.ds(off, tail)]],
                rd[0].at[pl.ds(0, tail)], sems_d[0])
            tcs.start()
            tcd.start()
            tcs.wait()
            tcd.wait()
            lax.fori_loop(0, tail // 8, block_body_for(0), perm_e + off,
                          unroll=1)

        pltpu.sync_copy(out_v, out_hbm.at[pl.ds(base, epw)])

    return sc_edge_kernel


@jax.jit
def kernel(x, edge_index, W1, b1, W2, b2):
    n, d = x.shape
    e_total = edge_index.shape[1]

    w1a = W1[:d]
    w1b = W1[d:]
    xs, xd = _project_nodes(x, w1a, w1b, b1.reshape(1, d))

    p = jnp.concatenate(
        [W2[:, 0], jnp.full((_L,), b2[0] / _L, dtype=jnp.float32)])

    sc_kernel = _make_sc_edge_kernel(e_total, d)
    return sc_kernel(xs, xd, edge_index, p)


# speedup vs baseline: 1.4901x; 1.0152x over previous
import functools

import jax
import jax.numpy as jnp
from jax import lax
from jax.experimental import pallas as pl
from jax.experimental.pallas import tpu as pltpu
from jax.experimental.pallas import tpu_sc as plsc

_NC = 2
_NS = 16
_NW = _NC * _NS
_L = 16

_CHUNK = 128


def _pack_bf16_pairs(a):
    h = a.shape[1] // 2
    lo = lax.bitcast_convert_type(
        a[:, :h].astype(jnp.bfloat16), jnp.uint16).astype(jnp.int32)
    hi = lax.bitcast_convert_type(
        a[:, h:].astype(jnp.bfloat16), jnp.uint16).astype(jnp.int32)
    return (hi << 16) | lo


def _proj_body(x_ref, w1_ref, b1_ref, w2_ref, b2_ref, xs_ref, xd_ref, p_ref):
    d = x_ref.shape[1]
    xv = x_ref[...]
    w1 = w1_ref[...]
    xs = jnp.dot(xv, w1[:d], preferred_element_type=jnp.float32)
    xd = (
        jnp.dot(xv, w1[d:], preferred_element_type=jnp.float32)
        + b1_ref[...]
    )
    xs_ref[...] = _pack_bf16_pairs(xs)
    xd_ref[...] = _pack_bf16_pairs(xd)

    @pl.when(pl.program_id(0) == 0)
    def _():
        p_ref[...] = jnp.concatenate(
            [w2_ref[...],
             jnp.full((1, _L), 1.0 / _L, jnp.float32) * b2_ref[0, 0]],
            axis=1,
        )


def _project_nodes(x, w1, b1_row, w2_row, b2_11):
    n, d = x.shape
    blk = 1000
    grid = n // blk
    return pl.pallas_call(
        _proj_body,
        grid=(grid,),
        in_specs=[
            pl.BlockSpec((blk, d), lambda i: (i, 0)),
            pl.BlockSpec((2 * d, d), lambda i: (0, 0)),
            pl.BlockSpec((1, d), lambda i: (0, 0)),
            pl.BlockSpec((1, d), lambda i: (0, 0)),
            pl.BlockSpec((1, 1), lambda i: (0, 0)),
        ],
        out_specs=[
            pl.BlockSpec((blk, d // 2), lambda i: (i, 0)),
            pl.BlockSpec((blk, d // 2), lambda i: (i, 0)),
            pl.BlockSpec((1, d + _L), lambda i: (0, 0)),
        ],
        out_shape=[
            jax.ShapeDtypeStruct((n, d // 2), jnp.int32),
            jax.ShapeDtypeStruct((n, d // 2), jnp.int32),
            jax.ShapeDtypeStruct((1, d + _L), jnp.float32),
        ],
    )(x, w1, b1_row, w2_row, b2_11)


def _make_sc_edge_kernel(e_total, d):
    epw = e_total // _NW
    nch = epw // _CHUNK
    tail = epw - nch * _CHUNK
    nwb = d // (2 * _L)
    dw = d // 2
    mesh = plsc.VectorSubcoreMesh(core_axis_name="c", subcore_axis_name="s")

    @functools.partial(
        pl.kernel,
        out_type=jax.ShapeDtypeStruct((e_total,), jnp.float32),
        mesh=mesh,
        scratch_types=[
            pltpu.VMEM((epw,), jnp.int32),
            pltpu.VMEM((epw,), jnp.int32),
            pltpu.VMEM((_CHUNK, dw), jnp.int32),
            pltpu.VMEM((_CHUNK, dw), jnp.int32),
            pltpu.VMEM((_CHUNK, dw), jnp.int32),
            pltpu.VMEM((_CHUNK, dw), jnp.int32),
            pltpu.VMEM((epw,), jnp.float32),
            pltpu.VMEM((d + _L,), jnp.float32),
            pltpu.SemaphoreType.DMA,
            pltpu.SemaphoreType.DMA,
            pltpu.SemaphoreType.DMA,
            pltpu.SemaphoreType.DMA,
        ],
        compiler_params=pltpu.CompilerParams(
            needs_layout_passes=False, use_tc_tiling_on_sc=False),
    )
    def sc_edge_kernel(xs_hbm, xd_hbm, ei_hbm, p_hbm, out_hbm,
                       idx_s, idx_d, rs0, rs1, rd0, rd1,
                       out_v, p_v, ss0, ss1, sd0, sd1):
        wid = lax.axis_index("s") * _NC + lax.axis_index("c")
        base = wid * epw

        pltpu.sync_copy(ei_hbm.at[0, pl.ds(base, epw)], idx_s)
        pltpu.sync_copy(ei_hbm.at[1, pl.ds(base, epw)], idx_d)
        pltpu.sync_copy(p_hbm.at[0], p_v)

        w2v = [p_v[pl.ds(_L * j, _L)] for j in range(2 * nwb)]
        bias = p_v[pl.ds(d, _L)]
        lane = lax.iota(jnp.int32, _L)
        xperm = {sh: jnp.bitwise_xor(lane, sh) for sh in (8, 4, 2, 1)}
        xmask = {sh: (lane & sh) == 0 for sh in (8, 4, 2, 1)}
        perm_e = (
            ((lane >> 3) & 1)
            | (((lane >> 2) & 1) << 1)
            | (((lane >> 1) & 1) << 2)
        )
        zero_v = jnp.zeros((_L,), jnp.float32)
        zero_b = jnp.zeros((2 * _L,), jnp.bfloat16)
        eight_v = jnp.full((_L,), 8, dtype=jnp.int32)
        himask = jnp.full((_L,), -65536, dtype=jnp.int32)

        def xg(v, sh):
            return v.at[xperm[sh]].get(mode="promise_in_bounds")

        rs = [rs0, rs1]
        rd = [rd0, rd1]
        sems_s = [ss0, ss1]
        sems_d = [sd0, sd1]
        nbuf = 2

        def gather(chunk, b):
            return (
                pltpu.make_async_copy(
                    xs_hbm.at[idx_s.at[pl.ds(chunk * _CHUNK, _CHUNK)]],
                    rs[b], sems_s[b]),
                pltpu.make_async_copy(
                    xd_hbm.at[idx_d.at[pl.ds(chunk * _CHUNK, _CHUNK)]],
                    rd[b], sems_d[b]),
            )

        def issue(chunk, b):
            cs, cd = gather(chunk, b)
            cs.start()
            cd.start()

        for b in range(nbuf):
            issue(b, b)

        def block_body_for(b):
            def block_body(k, pos):
                vs = []
                for t in range(8):
                    e = k * 8 + t
                    accs = [bias, zero_v]
                    for q in range(nwb):
                        ws = rs[b][e, pl.ds(_L * q, _L)]
                        wd = rd[b][e, pl.ds(_L * q, _L)]
                        bsum = (plsc.bitcast(ws, jnp.bfloat16)
                                + plsc.bitcast(wd, jnp.bfloat16))
                        brel = jnp.maximum(bsum, zero_b)
                        wi = plsc.bitcast(brel, jnp.int32)
                        r_lo = plsc.bitcast(wi << 16, jnp.float32)
                        r_hi = plsc.bitcast(wi & himask, jnp.float32)
                        accs[0] = accs[0] + r_lo * w2v[q]
                        accs[1] = accs[1] + r_hi * w2v[q + nwb]
                    vs.append(accs[0] + accs[1])
                for sh in (8, 4, 2):
                    m = xmask[sh]
                    vs = [
                        jnp.where(m, u + xg(u, sh), xg(v + xg(v, sh), sh))
                        for u, v in zip(vs[0::2], vs[1::2])
                    ]
                f = vs[0]
                f = f + xg(f, 1)
                plsc.store_scatter(out_v, [pos], f, mask=xmask[1])
                return pos + eight_v

            return block_body

        def consume(chunk, b, do_issue):
            cs, cd = gather(chunk, b)
            cs.wait()
            cd.wait()

            pos0 = perm_e + (chunk * _CHUNK)
            lax.fori_loop(0, _CHUNK // 8, block_body_for(b), pos0, unroll=1)

            if do_issue:
                @pl.when(chunk + nbuf < nch)
                def _(b=b, chunk=chunk):
                    issue(chunk + nbuf, b)

        def outer(i, carry):
            c4 = i * nbuf
            for b in range(nbuf):
                consume(c4 + b, b, True)
            return carry

        lax.fori_loop(0, nch // nbuf, outer, 0)
        for r in range(nch % nbuf):
            consume((nch // nbuf) * nbuf + r, r, False)

        if tail:
            off = nch * _CHUNK
            tcs = pltpu.make_async_copy(
                xs_hbm.at[idx_s.at[pl.ds(off, tail)]],
                rs[0].at[pl.ds(0, tail)], sems_s[0])
            tcd = pltpu.make_async_copy(
                xd_hbm.at[idx_d.at[pl.ds(off, tail)]],
                rd[0].at[pl.ds(0, tail)], sems_d[0])
            tcs.start()
            tcd.start()
            tcs.wait()
            tcd.wait()
            lax.fori_loop(0, tail // 8, block_body_for(0), perm_e + off,
                          unroll=1)

        pltpu.sync_copy(out_v, out_hbm.at[pl.ds(base, epw)])

    return sc_edge_kernel


@jax.jit
def kernel(x, edge_index, W1, b1, W2, b2):
    n, d = x.shape
    e_total = edge_index.shape[1]

    xs, xd, p = _project_nodes(
        x, W1, b1.reshape(1, d), W2.reshape(1, d), b2.reshape(1, 1))

    sc_kernel = _make_sc_edge_kernel(e_total, d)
    return sc_kernel(xs, xd, edge_index, p)
